# Initial kernel scaffold; baseline (speedup 1.0000x reference)
import jax
import jax.numpy as jnp
from jax.experimental import pallas as pl


def kernel(entity_table, W1, b1, W2, b2, gamma, beta, entity_ids, edge_index):
    raise NotImplementedError("write your pallas kernel here")


# speedup vs baseline: 9.4145x; 9.4145x over previous
import functools

import jax
import jax.numpy as jnp
from jax import lax
from jax.experimental import pallas as pl
from jax.experimental.pallas import tpu as pltpu
from jax.experimental.pallas import tpu_sc as plsc

F32 = jnp.float32
I32 = jnp.int32

NC = 2
NS = 16


@functools.lru_cache(maxsize=None)
def _build_gather_deg(num_ent, n, d, e):
    assert n % 8 == 0 and e % 128 == 0
    rows_chunk = 80
    n_chunks = n // rows_chunk
    xg_iters = -(-n_chunks // (NC * NS))
    npad = ((n + 639) // 640) * 640
    e_chunks = e // 128
    dg_iters = -(-e_chunks // NS)

    mesh = plsc.VectorSubcoreMesh(core_axis_name="c", subcore_axis_name="s",
                                  num_cores=NC, num_subcores=NS)

    @functools.partial(
        pl.kernel,
        mesh=mesh,
        out_type=(
            jax.ShapeDtypeStruct((n, d), F32),
            jax.ShapeDtypeStruct((npad,), F32),
        ),
        scratch_types=[
            pltpu.VMEM((rows_chunk,), I32),
            pltpu.VMEM((rows_chunk, d), F32),
            pltpu.VMEM((128,), I32),
            pltpu.VMEM((128,), F32),
            pltpu.VMEM_SHARED((npad,), F32),
            pltpu.SemaphoreType.DMA,
        ],
    )
    def k(tab_hbm, ids_hbm, dstz_hbm, zeros_hbm, ones_hbm,
          x_hbm, cnt_hbm,
          idbuf, rowbuf, dstbuf, onesbuf, cnt_sp, sem):
        c = lax.axis_index("c")
        s = lax.axis_index("s")
        w = s * NC + c

        def xg(j, carry):
            cid = w + NC * NS * j
            @pl.when(cid < n_chunks)
            def _():
                base = cid * rows_chunk
                pltpu.sync_copy(ids_hbm.at[pl.ds(base, rows_chunk)], idbuf)
                pltpu.async_copy(tab_hbm.at[idbuf], rowbuf, sem).wait()
                pltpu.sync_copy(rowbuf, x_hbm.at[pl.ds(base, rows_chunk)])
            return carry
        lax.fori_loop(0, xg_iters, xg, 0)

        @pl.when(c == 0)
        def _():
            pltpu.sync_copy(zeros_hbm, cnt_sp.at[pl.ds(s * 640, 640)])
            pltpu.sync_copy(ones_hbm, onesbuf)
            plsc.subcore_barrier()

            def dg(j, carry):
                cid = s + NS * j
                @pl.when(cid < e_chunks)
                def __():
                    pltpu.sync_copy(dstz_hbm.at[pl.ds(cid * 128, 128)], dstbuf)
                    pltpu.sync_copy(onesbuf, cnt_sp.at[dstbuf], add=True)
                return carry
            lax.fori_loop(0, dg_iters, dg, 0)

            plsc.subcore_barrier()
            pltpu.sync_copy(cnt_sp.at[pl.ds(s * 640, 640)],
                            cnt_hbm.at[pl.ds(s * 640, 640)])

    return k


@functools.lru_cache(maxsize=None)
def _build_message(n, h, e):
    assert e % 128 == 0 and n % NS == 0
    e_chunks = e // 128
    iters = -(-e_chunks // NS)
    rpt = (n // NS) & ~7
    tail = n - NS * rpt

    mesh = plsc.VectorSubcoreMesh(core_axis_name="c", subcore_axis_name="s",
                                  num_cores=NC, num_subcores=NS)

    @functools.partial(
        pl.kernel,
        mesh=mesh,
        out_type=(
            jax.ShapeDtypeStruct((n, h), F32),
            jax.ShapeDtypeStruct((n, h), F32),
        ),
        scratch_types=[
            pltpu.VMEM((128,), I32),
            pltpu.VMEM((128,), I32),
            pltpu.VMEM((128, h), F32),
            pltpu.VMEM_SHARED((n, h), F32),
            pltpu.SemaphoreType.DMA,
        ],
    )
    def k(ga_hbm, gb_hbm, src_hbm, dst_hbm, zrows_hbm,
          acca_hbm, accb_hbm,
          sidx, didx, rowbuf, acc_sp, sem):
        c = lax.axis_index("c")
        s = lax.axis_index("s")
        r0 = s * rpt

        pltpu.sync_copy(zrows_hbm.at[pl.ds(0, rpt)], acc_sp.at[pl.ds(r0, rpt)])
        @pl.when(s == NS - 1)
        def _():
            pltpu.sync_copy(zrows_hbm.at[pl.ds(0, tail)],
                            acc_sp.at[pl.ds(NS * rpt, tail)])
        plsc.subcore_barrier()

        def run(g_hbm):
            def it(j, carry):
                cid = s + NS * j
                @pl.when(cid < e_chunks)
                def __():
                    base = cid * 128
                    pltpu.sync_copy(src_hbm.at[pl.ds(base, 128)], sidx)
                    pltpu.sync_copy(dst_hbm.at[pl.ds(base, 128)], didx)
                    pltpu.async_copy(g_hbm.at[sidx], rowbuf, sem).wait()
                    pltpu.sync_copy(rowbuf, acc_sp.at[didx], add=True)
                return carry
            lax.fori_loop(0, iters, it, 0)

        @pl.when(c == 0)
        def _():
            run(ga_hbm)
        @pl.when(c == 1)
        def _():
            run(gb_hbm)

        plsc.subcore_barrier()

        def drain(out_hbm):
            sl = pl.ds(r0, rpt)
            pltpu.sync_copy(acc_sp.at[sl], out_hbm.at[sl])
            @pl.when(s == NS - 1)
            def _():
                tl = pl.ds(NS * rpt, tail)
                pltpu.sync_copy(acc_sp.at[tl], out_hbm.at[tl])

        @pl.when(c == 0)
        def _():
            drain(acca_hbm)
        @pl.when(c == 1)
        def _():
            drain(accb_hbm)

    return k


_RB = 1000


def _tc_scale_matmul_body(x_ref, w_ref, cnt_ref, ga_ref, gb_ref, dinv_ref):
    xb = x_ref[...]
    hh = jnp.dot(xb, w_ref[...], preferred_element_type=F32)
    dinv = lax.rsqrt(cnt_ref[...] + 1.0)
    g = hh * dinv
    half = g.shape[1] // 2
    ga_ref[...] = g[:, :half]
    gb_ref[...] = g[:, half:]
    dinv_ref[...] = dinv


def _tc1_call(x, w1, cnt):
    n, d = x.shape
    h = d // 2
    grid = (n // _RB,)
    return pl.pallas_call(
        _tc_scale_matmul_body,
        grid=grid,
        in_specs=[
            pl.BlockSpec((_RB, d), lambda i: (i, 0)),
            pl.BlockSpec((d, d), lambda i: (0, 0)),
            pl.BlockSpec((_RB, 1), lambda i: (i, 0)),
        ],
        out_specs=[
            pl.BlockSpec((_RB, h), lambda i: (i, 0)),
            pl.BlockSpec((_RB, h), lambda i: (i, 0)),
            pl.BlockSpec((_RB, 1), lambda i: (i, 0)),
        ],
        out_shape=[
            jax.ShapeDtypeStruct((n, h), F32),
            jax.ShapeDtypeStruct((n, h), F32),
            jax.ShapeDtypeStruct((n, 1), F32),
        ],
    )(x, w1, cnt)


def _post_conv(acc_a, acc_b, g_a, g_b, xres, dinv, b, gamma, beta):
    acc = jnp.concatenate([acc_a, acc_b], axis=1)
    g = jnp.concatenate([g_a, g_b], axis=1)
    conv = dinv * (acc + g) + b
    z = jnp.maximum(conv, 0.0) + xres
    mu = jnp.mean(z, axis=1, keepdims=True)
    var = jnp.mean((z - mu) ** 2, axis=1, keepdims=True)
    return (z - mu) * lax.rsqrt(var + 1e-5) * gamma + beta


def _tc_mid_body(acca_ref, accb_ref, ga_ref, gb_ref, x_ref, dinv_ref,
                 b_ref, gm_ref, bt_ref, w2_ref,
                 x2_ref, g2a_ref, g2b_ref):
    dinv = dinv_ref[...]
    xn = _post_conv(acca_ref[...], accb_ref[...], ga_ref[...], gb_ref[...],
                    x_ref[...], dinv, b_ref[...], gm_ref[...], bt_ref[...])
    x2_ref[...] = xn
    h2 = jnp.dot(xn, w2_ref[...], preferred_element_type=F32)
    g2 = h2 * dinv
    half = g2.shape[1] // 2
    g2a_ref[...] = g2[:, :half]
    g2b_ref[...] = g2[:, half:]


def _tc2_call(acc_a, acc_b, g_a, g_b, x, dinv, b1, gamma, beta, w2):
    n, h = acc_a.shape
    d = 2 * h
    grid = (n // _RB,)
    bs_h = pl.BlockSpec((_RB, h), lambda i: (i, 0))
    bs_d = pl.BlockSpec((_RB, d), lambda i: (i, 0))
    bs_1 = pl.BlockSpec((_RB, 1), lambda i: (i, 0))
    bs_v = pl.BlockSpec((1, d), lambda i: (0, 0))
    return pl.pallas_call(
        _tc_mid_body,
        grid=grid,
        in_specs=[bs_h, bs_h, bs_h, bs_h, bs_d, bs_1, bs_v, bs_v, bs_v,
                  pl.BlockSpec((d, d), lambda i: (0, 0))],
        out_specs=[bs_d, bs_h, bs_h],
        out_shape=[
            jax.ShapeDtypeStruct((n, d), F32),
            jax.ShapeDtypeStruct((n, h), F32),
            jax.ShapeDtypeStruct((n, h), F32),
        ],
    )(acc_a, acc_b, g_a, g_b, x, dinv, b1, gamma, beta, w2)


def _tc_final_body(acca_ref, accb_ref, ga_ref, gb_ref, x_ref, dinv_ref,
                   b_ref, gm_ref, bt_ref, out_ref):
    out_ref[...] = _post_conv(acca_ref[...], accb_ref[...], ga_ref[...],
                              gb_ref[...], x_ref[...], dinv_ref[...],
                              b_ref[...], gm_ref[...], bt_ref[...])


def _tc3_call(acc_a, acc_b, g_a, g_b, x, dinv, b2, gamma, beta):
    n, h = acc_a.shape
    d = 2 * h
    grid = (n // _RB,)
    bs_h = pl.BlockSpec((_RB, h), lambda i: (i, 0))
    bs_d = pl.BlockSpec((_RB, d), lambda i: (i, 0))
    bs_1 = pl.BlockSpec((_RB, 1), lambda i: (i, 0))
    bs_v = pl.BlockSpec((1, d), lambda i: (0, 0))
    return pl.pallas_call(
        _tc_final_body,
        grid=grid,
        in_specs=[bs_h, bs_h, bs_h, bs_h, bs_d, bs_1, bs_v, bs_v, bs_v],
        out_specs=bs_d,
        out_shape=jax.ShapeDtypeStruct((n, d), F32),
    )(acc_a, acc_b, g_a, g_b, x, dinv, b2, gamma, beta)


def kernel(entity_table, W1, b1, W2, b2, gamma, beta, entity_ids, edge_index):
    num_ent, d = entity_table.shape
    n = entity_ids.shape[0]
    e = edge_index.shape[1]
    h = d // 2

    src = edge_index[0].astype(I32)
    dst = edge_index[1].astype(I32)
    ids = entity_ids.astype(I32)

    zeros640 = jnp.zeros((640,), F32)
    ones128 = jnp.ones((128,), F32)
    zrows = jnp.zeros((((n // NS) + 7) & ~7, h), F32)

    x, cnt = _build_gather_deg(num_ent, n, d, e)(
        entity_table, ids, dst, zeros640, ones128)
    cnt = cnt[:n].reshape(n, 1)

    b1r = b1.reshape(1, d)
    b2r = b2.reshape(1, d)
    gmr = gamma.reshape(1, d)
    btr = beta.reshape(1, d)

    g1a, g1b, dinv = _tc1_call(x, W1, cnt)

    msg = _build_message(n, h, e)
    acc1a, acc1b = msg(g1a, g1b, src, dst, zrows)

    x2, g2a, g2b = _tc2_call(acc1a, acc1b, g1a, g1b, x, dinv,
                             b1r, gmr, btr, W2)

    acc2a, acc2b = msg(g2a, g2b, src, dst, zrows)

    out = _tc3_call(acc2a, acc2b, g2a, g2b, x2, dinv, b2r, gmr, btr)
    return out
